# baseline (speedup 1.0000x reference)
import functools

import jax
import jax.numpy as jnp
import numpy as np
from jax import lax
from jax.experimental import pallas as pl
from jax.experimental.pallas import tpu as pltpu
from jax.experimental.pallas import tpu_sc as plsc

KS = 128
GF = 0.1

_BYTE_SH = [0xFF, 0xFF00, 0xFF0000, -16777216]


def _step_table(length: int) -> np.ndarray:
    vals = []
    for ml in range(length + 1):
        max_tokens = max(1, int(round(GF * ml)))
        vals.append(max(1, int(round(ml / max_tokens))))
    return np.asarray(vals, dtype=np.int32)


def _make_sc_kernel(length: int, nb: int, table_len: int):
    lw = length // 4
    lwp = lw + 16
    nw = 32
    rows_per_w = length // nw
    nbuf = 8
    iters = rows_per_w // nbuf
    mesh = plsc.VectorSubcoreMesh(core_axis_name="c", subcore_axis_name="s")

    @functools.partial(
        pl.kernel,
        mesh=mesh,
        out_type=jax.ShapeDtypeStruct((length, lw), jnp.int32),
        scratch_types=[
            pltpu.VMEM((16,), jnp.int32),
            pltpu.VMEM((table_len,), jnp.int32),
            pltpu.VMEM((lwp,), jnp.int32),
            pltpu.VMEM((lw,), jnp.int32),
        ]
        + [pltpu.VMEM((lwp,), jnp.int32) for _ in range(nbuf)]
        + [pltpu.SemaphoreType.DMA for _ in range(nbuf)],
        compiler_params=pltpu.CompilerParams(needs_layout_passes=False),
    )
    def sc_kernel(seq_hbm, table_hbm, out_hbm, seqb, tabb, tmpl, zbuf, *rest):
        rowbufs = rest[:nbuf]
        sems = rest[nbuf:]

        wid = lax.axis_index("c") * 16 + lax.axis_index("s")
        r0 = wid * rows_per_w

        pltpu.sync_copy(seq_hbm, seqb)
        pltpu.sync_copy(table_hbm, tabb)

        sv = seqb[pl.ds(0, 16)]
        max_len = sv[0]
        for b in range(1, nb):
            max_len = jnp.maximum(max_len, sv[b])
        step = plsc.load_gather(tabb, [jnp.full((16,), max_len, jnp.int32)])[0]

        lane = lax.broadcasted_iota(jnp.int32, (16,), 0)

        def build_tmpl(g, _):
            w16 = g * 16 + lane
            acc = jnp.zeros((16,), jnp.int32)
            for byte in range(4):
                pos = w16 * 4 + byte
                ng = jnp.logical_or(pos >= max_len, pos % step != 0)
                acc = acc | jnp.where(ng, _BYTE_SH[byte], 0)
            tmpl[pl.ds(g * 16, 16)] = acc
            zbuf[pl.ds(g * 16, 16)] = jnp.zeros((16,), jnp.int32)
            return 0

        lax.fori_loop(0, lw // 16, build_tmpl, 0)
        tmpl[pl.ds(lw, 16)] = jnp.zeros((16,), jnp.int32)

        def is_glob(r):
            return jnp.logical_and(r < max_len, r % step == 0)

        def fire(b, r):
            @pl.when(is_glob(r))
            def _():
                pltpu.async_copy(zbuf, out_hbm.at[r], sems[b])

            @pl.when(jnp.logical_not(is_glob(r)))
            def _():
                pltpu.async_copy(
                    rowbufs[b].at[pl.ds(0, lw)], out_hbm.at[r], sems[b])

        for b in range(nbuf):
            buf = rowbufs[b]
            rb0 = r0 + b
            lo = jnp.maximum(0, rb0 - KS)

            def init_g(g, _, buf=buf, lo=lo, rb0=rb0):
                w16 = g * 16 + lane
                tv = tmpl[pl.ds(g * 16, 16)]
                m = jnp.zeros((16,), jnp.int32)
                for byte in range(4):
                    pos = w16 * 4 + byte
                    inwin = jnp.logical_and(pos >= lo, pos <= rb0)
                    m = m | jnp.where(inwin, _BYTE_SH[byte], 0)
                buf[pl.ds(g * 16, 16)] = tv & ~m
                return 0

            lax.fori_loop(0, lwp // 16, init_g, 0)
            fire(b, rb0)

        def body(k, _):
            for b in range(nbuf):
                buf = rowbufs[b]
                r = r0 + b + 8 * k
                pltpu.make_async_copy(
                    buf.at[pl.ds(0, lw)], out_hbm.at[r0], sems[b]).wait()

                p = b % 4
                q = (b + 1) % 4

                @pl.when(r >= 136)
                def _(buf=buf, r=r, p=p):
                    idx = ((r - 136) >> 2) + lane
                    tv = plsc.load_gather(tmpl, [idx])
                    v = plsc.load_gather(buf, [idx])
                    if p == 0:
                        mm = jnp.where(lane <= 1, -1, 0)
                    else:
                        k0 = (1 << (8 * p)) - 1
                        mm = jnp.where(lane == 0, ~k0,
                                       jnp.where(lane == 1, -1,
                                                 jnp.where(lane == 2, k0, 0)))
                    plsc.store_scatter(buf, [idx], (v & ~mm) | (tv & mm))

                idxz = ((r - 7) >> 2) + lane
                v = plsc.load_gather(buf, [idxz])
                if q == 0:
                    mz = jnp.where(lane <= 1, -1, 0)
                else:
                    kq = (1 << (8 * q)) - 1
                    ke = (1 << (8 * (p + 1))) - 1
                    mz = jnp.where(lane == 0, ~kq,
                                   jnp.where(lane == 1, -1,
                                             jnp.where(lane == 2, ke, 0)))
                plsc.store_scatter(buf, [idxz], v & ~mz)

                fire(b, r)
            return 0

        lax.fori_loop(1, iters, body, 0)

        for b in range(nbuf):
            pltpu.make_async_copy(
                rowbufs[b].at[pl.ds(0, lw)], out_hbm.at[r0], sems[b]).wait()

    return sc_kernel


def kernel(x, timestamps, seq_lens):
    length = x.shape[1]
    nb = seq_lens.shape[0]
    table = _step_table(length)
    table_len = ((len(table) + 15) // 16) * 16 + 16
    table_pad = np.zeros((table_len,), np.int32)
    table_pad[: len(table)] = table

    seq_pad = jnp.zeros((16,), jnp.int32).at[:nb].set(seq_lens.astype(jnp.int32))

    sc = _make_sc_kernel(length, nb, table_len)
    out_words = sc(seq_pad, jnp.asarray(table_pad))

    mask = out_words.view(jnp.uint8).astype(jnp.bool_)
    one = (seq_lens[0] * 0 + 1).astype(x.dtype)
    return (x * one, timestamps * one.astype(timestamps.dtype), mask)
